# baseline (speedup 1.0000x reference)
import functools

import jax
import jax.numpy as jnp
from jax import lax
from jax.experimental import pallas as pl
from jax.experimental.pallas import tpu as pltpu
from jax.experimental.pallas import tpu_sc as plsc

_K = 5
_CHUNK = 64
_PD = 128


@functools.lru_cache(maxsize=None)
def _make_gather(n_total, n_var, d_model):
    info = plsc.get_sparse_core_info()
    nc, ns = info.num_cores, info.num_subcores
    nw = nc * ns

    chunk = _CHUNK
    per_w = n_total // nw
    n_chunks = per_w // chunk
    n_groups = n_chunks // _K
    assert per_w * nw == n_total and n_groups * _K * chunk == per_w

    mesh = plsc.VectorSubcoreMesh(core_axis_name="c", subcore_axis_name="s")

    @functools.partial(
        pl.kernel,
        mesh=mesh,
        out_type=jax.ShapeDtypeStruct((n_total, _PD), jnp.float32),
        scratch_types=[pltpu.VMEM((chunk,), jnp.int32) for _ in range(_K)]
        + [
            pltpu.VMEM((_K, chunk, _PD), jnp.float32),
            pltpu.VMEM_SHARED((n_var, _PD), jnp.float32),
        ]
        + [pltpu.SemaphoreType.DMA for _ in range(3 * _K)],
    )
    def gather_kernel(idx_hbm, table_hbm, out_hbm, *refs):
        idx_v = refs[0:_K]
        rows_v, table_s = refs[_K], refs[_K + 1]
        sems = refs[_K + 2 :]
        isem = sems[0:_K]
        gsem = sems[_K : 2 * _K]
        osem = sems[2 * _K : 3 * _K]

        sid = lax.axis_index("s")
        wid = sid * nc + lax.axis_index("c")
        base = wid * per_w

        @pl.when(sid == 0)
        def _stage():
            pltpu.sync_copy(table_hbm, table_s)

        plsc.subcore_barrier()

        def idx_src(j):
            return idx_hbm.at[pl.ds(base + j * chunk, chunk)]

        def out_dst(j):
            return out_hbm.at[pl.ds(base + j * chunk, chunk)]

        def group(g, carry):
            i0 = g * _K
            for b in range(_K):
                pltpu.async_copy(idx_src(i0 + b), idx_v[b], isem[b])
            for b in range(_K):
                pltpu.make_async_copy(idx_src(i0 + b), idx_v[b], isem[b]).wait()
                pltpu.async_copy(table_s.at[idx_v[b]], rows_v.at[b], gsem[b])
            for b in range(_K):
                pltpu.make_async_copy(
                    table_s.at[idx_v[b]], rows_v.at[b], gsem[b]
                ).wait()
                pltpu.async_copy(rows_v.at[b], out_dst(i0 + b), osem[b])
            for b in range(_K):
                pltpu.make_async_copy(
                    rows_v.at[b], out_dst(i0 + b), osem[b]
                ).wait()
            return carry

        lax.fori_loop(0, n_groups, group, 0)

    return gather_kernel


def kernel(x, emb_table):
    b, s = x.shape
    v, d = emb_table.shape
    tab = jnp.pad(emb_table, ((0, 0), (0, _PD - d)))
    out2d = _make_gather(b * s, v, d)(x.astype(jnp.int32).reshape(-1), tab)
    return out2d[:, :d].reshape(b, s, d)
